# baseline (speedup 1.0000x reference)
import functools

import jax
import jax.numpy as jnp
from jax import lax
from jax.experimental import pallas as pl
from jax.experimental.pallas import tpu as pltpu
from jax.experimental.pallas import tpu_sc as plsc

_N = 1000
_B = 256
_ROW = 256 * 256
_TOTAL = _N * _ROW
_BOUND = _B * _ROW
_NW = 32
_W_COMP = 8
_W_MEM = _NW - _W_COMP
_SZ_COMP = _BOUND // _W_COMP
_SZ_MEM = (_TOTAL - _BOUND) // _W_MEM
_CHUNK = 32768
_NC_COMP = _SZ_COMP // _CHUNK
_NC_MEM = _SZ_MEM // _CHUNK
_NBUF = 3


def _stream_range(src_hbm, out_hbm, base, nchunks, bufs, rsems, wsems):
    def rd(i, s):
        return pltpu.make_async_copy(
            src_hbm.at[pl.ds(base + i * _CHUNK, _CHUNK)], bufs[s], rsems[s])

    def wr(i, s):
        return pltpu.make_async_copy(
            bufs[s], out_hbm.at[pl.ds(base + i * _CHUNK, _CHUNK)], wsems[s])

    rd(0, 0).start()
    for i in range(nchunks):
        s = i % _NBUF
        rd(i, s).wait()
        wr(i, s).start()
        ni = i + 1
        if ni < nchunks:
            ns = ni % _NBUF
            if ni >= _NBUF:
                wr(ni - _NBUF, ns).wait()
            rd(ni, ns).start()
    for i in range(max(nchunks - _NBUF, 0), nchunks):
        wr(i, i % _NBUF).wait()


def _enqueue_body(comp_hbm, mem_hbm, out_hbm,
                  buf0, buf1, buf2, rs0, rs1, rs2, ws0, ws1, ws2):
    wid = lax.axis_index("s") * 2 + lax.axis_index("c")
    bufs = (buf0, buf1, buf2)
    rsems = (rs0, rs1, rs2)
    wsems = (ws0, ws1, ws2)

    @pl.when(wid < _W_COMP)
    def _():
        _stream_range(comp_hbm, out_hbm, wid * _SZ_COMP, _NC_COMP,
                      bufs, rsems, wsems)

    @pl.when(wid >= _W_COMP)
    def _():
        base = _BOUND + (wid - _W_COMP) * _SZ_MEM
        _stream_range(mem_hbm, out_hbm, base, _NC_MEM, bufs, rsems, wsems)


def kernel(memory_bank, components):
    comps = jax.lax.stop_gradient(components)
    mesh = plsc.VectorSubcoreMesh(core_axis_name="c", subcore_axis_name="s")
    run = functools.partial(
        pl.kernel,
        out_type=jax.ShapeDtypeStruct((_TOTAL,), jnp.float32),
        mesh=mesh,
        scratch_types=(
            [pltpu.VMEM((_CHUNK,), jnp.float32)] * _NBUF
            + [pltpu.SemaphoreType.DMA] * (2 * _NBUF)
        ),
    )(_enqueue_body)
    flat = run(comps.reshape(_BOUND), memory_bank.reshape(_TOTAL))
    return flat.reshape(_N, 256, 256)
